# baseline (speedup 1.0000x reference)
import jax
import jax.numpy as jnp
from jax import lax
from jax.experimental import pallas as pl
from jax.experimental.pallas import tpu as pltpu
from jax.experimental.pallas import tpu_sc as plsc

_N = 10000
_E = 320000
_H = 128
_G = 64
_C = 8

_NC = 2
_NS = 16
_NW = _NC * _NS
_WIN = 125
_WPW = _E // (_NW * _WIN)
_HPW = _WPW // 2
_CH = 2000
_NCH = _N // _CH
_TCH = (_NCH + _NS - 1) // _NS
_DW = _H


def _zero_acc(zero_hbm, acc, sid, sem):
    @pl.loop(0, _TCH)
    def _zero(t):
        j = sid + t * _NS

        @pl.when(j < _NCH)
        def _():
            r0 = pl.multiple_of(j * _CH, 8)
            pltpu.async_copy(zero_hbm, acc.at[pl.ds(r0, _CH)], sem)

    @pl.loop(0, _TCH)
    def _zwait(t):
        j = sid + t * _NS

        @pl.when(j < _NCH)
        def _():
            r0 = pl.multiple_of(j * _CH, 8)
            pltpu.make_async_copy(zero_hbm, acc.at[pl.ds(r0, _CH)], sem).wait()


def _write_partial(acc, out_hbm, cid, sid, sem):
    @pl.loop(0, _TCH)
    def _out(t):
        j = sid + t * _NS

        @pl.when(j < _NCH)
        def _():
            r0 = pl.multiple_of(j * _CH, 8)
            pltpu.async_copy(acc.at[pl.ds(r0, _CH)],
                             out_hbm.at[pl.ds(cid * _N + r0, _CH)], sem)

    @pl.loop(0, _TCH)
    def _owait(t):
        j = sid + t * _NS

        @pl.when(j < _NCH)
        def _():
            r0 = pl.multiple_of(j * _CH, 8)
            pltpu.make_async_copy(acc.at[pl.ds(r0, _CH)],
                                  out_hbm.at[pl.ds(cid * _N + r0, _CH)],
                                  sem).wait()


def _sc_scatter_body(z_hbm, src_hbm, dst_hbm, zero_hbm, out_hbm,
                     sidx, didx, rows0, rows1, acc,
                     gsem0, gsem1, zsem):
    cid = lax.axis_index("c")
    sid = lax.axis_index("s")
    wid = sid * _NC + cid
    base = pl.multiple_of(wid * _WPW, 8)

    pltpu.async_copy(src_hbm.at[pl.ds(base, _HPW)], sidx, gsem0)
    pltpu.async_copy(dst_hbm.at[pl.ds(base, _HPW)], didx, gsem1)

    _zero_acc(zero_hbm, acc, sid, zsem)
    pltpu.make_async_copy(src_hbm.at[pl.ds(base, _HPW)], sidx, gsem0).wait()
    pltpu.make_async_copy(dst_hbm.at[pl.ds(base, _HPW)], didx, gsem1).wait()
    plsc.subcore_barrier()

    pltpu.async_copy(z_hbm.at[sidx.at[0]], rows0, gsem0)
    pltpu.async_copy(z_hbm.at[sidx.at[1]], rows1, gsem1)

    @pl.loop(0, _WPW // 2)
    def _edges(t):
        j = 2 * t

        @pl.when(t == _HPW // 2)
        def _():
            pltpu.sync_copy(src_hbm.at[pl.ds(base + _HPW, _HPW)], sidx)
            pltpu.sync_copy(dst_hbm.at[pl.ds(base + _HPW, _HPW)], didx)
            pltpu.async_copy(z_hbm.at[sidx.at[0]], rows0, gsem0)
            pltpu.async_copy(z_hbm.at[sidx.at[1]], rows1, gsem1)

        jl = lax.rem(j, _HPW)
        not_last = jnp.logical_and(t != _HPW // 2 - 1, t != _WPW // 2 - 1)

        pltpu.make_async_copy(z_hbm.at[sidx.at[jl]], rows0, gsem0).wait()
        pltpu.sync_copy(rows0, acc.at[didx.at[jl]], add=True)

        @pl.when(not_last)
        def _():
            pltpu.async_copy(z_hbm.at[sidx.at[jl + 2]], rows0, gsem0)

        pltpu.make_async_copy(z_hbm.at[sidx.at[jl + 1]], rows1, gsem1).wait()
        pltpu.sync_copy(rows1, acc.at[didx.at[jl + 1]], add=True)

        @pl.when(not_last)
        def _():
            pltpu.async_copy(z_hbm.at[sidx.at[jl + 3]], rows1, gsem1)

    plsc.subcore_barrier()
    _write_partial(acc, out_hbm, cid, sid, gsem0)


def _sc_scatter(z, src2, dst2, zero_chunk):
    f = pl.kernel(
        _sc_scatter_body,
        out_type=jax.ShapeDtypeStruct((2 * _N, _H), jnp.float32),
        mesh=plsc.VectorSubcoreMesh(core_axis_name="c", subcore_axis_name="s"),
        scratch_types=[
            pltpu.VMEM((_HPW, _WIN), jnp.int32),
            pltpu.VMEM((_HPW, _WIN), jnp.int32),
            pltpu.VMEM((_WIN, _H), jnp.float32),
            pltpu.VMEM((_WIN, _H), jnp.float32),
            pltpu.VMEM_SHARED((_N, _H), jnp.float32),
            pltpu.SemaphoreType.DMA,
            pltpu.SemaphoreType.DMA,
            pltpu.SemaphoreType.DMA,
        ],
    )
    return f(z, src2, dst2, zero_chunk)


def _sc_degree_body(dst_hbm, ones_hbm, zero_hbm, out_hbm,
                    didx, ones_v, acc, sem0, zsem):
    cid = lax.axis_index("c")
    sid = lax.axis_index("s")
    wid = sid * _NC + cid
    base = pl.multiple_of(wid * _WPW, 8)

    pltpu.async_copy(dst_hbm.at[pl.ds(base, _WPW)], didx, sem0)
    pltpu.sync_copy(ones_hbm, ones_v)

    _zero_acc(zero_hbm, acc, sid, zsem)
    pltpu.make_async_copy(dst_hbm.at[pl.ds(base, _WPW)], didx, sem0).wait()
    plsc.subcore_barrier()

    @pl.loop(0, _WPW // 16)
    def _edges(b):
        hs = [pltpu.async_copy(ones_v, acc.at[didx.at[b * 16 + k]], zsem,
                               add=True)
              for k in range(16)]
        for h in hs:
            h.wait()

    plsc.subcore_barrier()
    _write_partial(acc, out_hbm, cid, sid, sem0)


def _sc_degree(dst2, ones_win, zero_chunk):
    f = pl.kernel(
        _sc_degree_body,
        out_type=jax.ShapeDtypeStruct((2 * _N, _DW), jnp.float32),
        mesh=plsc.VectorSubcoreMesh(core_axis_name="c", subcore_axis_name="s"),
        scratch_types=[
            pltpu.VMEM((_WPW, _WIN), jnp.int32),
            pltpu.VMEM((_WIN, _DW), jnp.float32),
            pltpu.VMEM_SHARED((_N, _DW), jnp.float32),
            pltpu.SemaphoreType.DMA,
            pltpu.SemaphoreType.DMA,
        ],
    )
    return f(dst2, ones_win, zero_chunk)


_R = 2000


def _tmm_body(x_ref, w_ref, y_ref):
    y_ref[...] = jnp.dot(x_ref[...], w_ref[...],
                         preferred_element_type=jnp.float32)


def _tc_matmul(x, w0):
    return pl.pallas_call(
        _tmm_body,
        grid=(_N // _R,),
        in_specs=[
            pl.BlockSpec((_R, _H), lambda i: (i, 0)),
            pl.BlockSpec((_H, _H), lambda i: (0, 0)),
        ],
        out_specs=pl.BlockSpec((_R, _H), lambda i: (i, 0)),
        out_shape=jax.ShapeDtypeStruct((_N, _H), jnp.float32),
    )(x, w0)


_NB = _N // _R


def _t0_body(y_ref, d0_ref, d1_ref, z_ref, dinv_ref):
    d = d0_ref[...][:, :1] + d1_ref[...][:, :1]
    dinv = jnp.broadcast_to(lax.rsqrt(1.0 + d), (_R, _H))
    dinv_ref[...] = dinv
    z_ref[...] = y_ref[...] * dinv


def _tc_first(y, degp):
    return pl.pallas_call(
        _t0_body,
        grid=(_NB,),
        in_specs=[
            pl.BlockSpec((_R, _H), lambda i: (i, 0)),
            pl.BlockSpec((_R, _DW), lambda i: (i, 0)),
            pl.BlockSpec((_R, _DW), lambda i: (i + _NB, 0)),
        ],
        out_specs=[
            pl.BlockSpec((_R, _H), lambda i: (i, 0)),
            pl.BlockSpec((_R, _H), lambda i: (i, 0)),
        ],
        out_shape=[
            jax.ShapeDtypeStruct((_N, _H), jnp.float32),
            jax.ShapeDtypeStruct((_N, _H), jnp.float32),
        ],
    )(y, degp, degp)


def _tmid_body(p0_ref, p1_ref, z_ref, dinv_ref, b_ref, w_ref, zo_ref):
    dinv = dinv_ref[...]
    h = jnp.maximum(
        dinv * (p0_ref[...] + p1_ref[...] + z_ref[...]) + b_ref[...], 0.0)
    zo_ref[...] = jnp.dot(h, w_ref[...],
                          preferred_element_type=jnp.float32) * dinv


def _tc_mid(p, z, dinv, b, w):
    return pl.pallas_call(
        _tmid_body,
        grid=(_NB,),
        in_specs=[
            pl.BlockSpec((_R, _H), lambda i: (i, 0)),
            pl.BlockSpec((_R, _H), lambda i: (i + _NB, 0)),
            pl.BlockSpec((_R, _H), lambda i: (i, 0)),
            pl.BlockSpec((_R, _H), lambda i: (i, 0)),
            pl.BlockSpec((1, _H), lambda i: (0, 0)),
            pl.BlockSpec((_H, _H), lambda i: (0, 0)),
        ],
        out_specs=pl.BlockSpec((_R, _H), lambda i: (i, 0)),
        out_shape=jax.ShapeDtypeStruct((_N, _H), jnp.float32),
    )(p, p, z, dinv, b, w)


def _t4_body(p0_ref, p1_ref, z_ref, dinv_ref, b_ref, batch_ref, wc_ref,
             bc_ref, sums_ref, cnt_ref, out_ref):
    i = pl.program_id(0)
    h = dinv_ref[...] * (p0_ref[...] + p1_ref[...] + z_ref[...]) + b_ref[...]
    gids = lax.broadcasted_iota(jnp.int32, (_R, _G), 1)
    m = (batch_ref[...] == gids).astype(jnp.float32)
    s = lax.dot_general(m, h, (((0,), (0,)), ((), ())),
                        preferred_element_type=jnp.float32)
    c = jnp.broadcast_to(jnp.sum(m, axis=0)[:, None], (_G, _H))

    @pl.when(i == 0)
    def _():
        sums_ref[...] = s
        cnt_ref[...] = c

    @pl.when(i > 0)
    def _():
        sums_ref[...] += s
        cnt_ref[...] += c

    @pl.when(i == _NB - 1)
    def _():
        mean = sums_ref[...] / jnp.maximum(cnt_ref[...], 1.0)
        out_ref[...] = jnp.dot(mean, wc_ref[...],
                               preferred_element_type=jnp.float32) + bc_ref[...]


def _tc_pool(p, z, dinv, b, batch2d, wc_pad, bc_pad):
    outs = pl.pallas_call(
        _t4_body,
        grid=(_NB,),
        in_specs=[
            pl.BlockSpec((_R, _H), lambda i: (i, 0)),
            pl.BlockSpec((_R, _H), lambda i: (i + _NB, 0)),
            pl.BlockSpec((_R, _H), lambda i: (i, 0)),
            pl.BlockSpec((_R, _H), lambda i: (i, 0)),
            pl.BlockSpec((1, _H), lambda i: (0, 0)),
            pl.BlockSpec((_R, 1), lambda i: (i, 0)),
            pl.BlockSpec((_H, _H), lambda i: (0, 0)),
            pl.BlockSpec((1, _H), lambda i: (0, 0)),
        ],
        out_specs=[
            pl.BlockSpec((_G, _H), lambda i: (0, 0)),
            pl.BlockSpec((_G, _H), lambda i: (0, 0)),
            pl.BlockSpec((_G, _H), lambda i: (0, 0)),
        ],
        out_shape=[
            jax.ShapeDtypeStruct((_G, _H), jnp.float32),
            jax.ShapeDtypeStruct((_G, _H), jnp.float32),
            jax.ShapeDtypeStruct((_G, _H), jnp.float32),
        ],
    )(p, p, z, dinv, b, batch2d, wc_pad, bc_pad)
    return outs[2]


def kernel(x, edge_index, batch, W0, b0, W1, b1, W2, b2, W3, b3, Wc, bc):
    src2 = edge_index[0].reshape(_E // _WIN, _WIN)
    dst2 = edge_index[1].reshape(_E // _WIN, _WIN)
    zero_chunk = jnp.zeros((_CH, _H), jnp.float32)
    ones_win = jnp.ones((_WIN, _DW), jnp.float32)

    degp = _sc_degree(dst2, ones_win, jnp.zeros((_CH, _DW), jnp.float32))
    y0 = _tc_matmul(x, W0)

    z, dinv = _tc_first(y0, degp)

    for b, w in ((b0, W1), (b1, W2), (b2, W3)):
        p = _sc_scatter(z, src2, dst2, zero_chunk)
        z = _tc_mid(p, z, dinv, b.reshape(1, _H), w)

    p = _sc_scatter(z, src2, dst2, zero_chunk)

    wc_pad = jnp.zeros((_H, _H), jnp.float32).at[:, :_C].set(Wc)
    bc_pad = jnp.zeros((1, _H), jnp.float32).at[0, :_C].set(bc)
    out = _tc_pool(p, z, dinv, b3.reshape(1, _H),
                   batch.reshape(_N, 1), wc_pad, bc_pad)
    return out[:, :_C]


# speedup vs baseline: 1.0077x; 1.0077x over previous
import jax
import jax.numpy as jnp
from jax import lax
from jax.experimental import pallas as pl
from jax.experimental.pallas import tpu as pltpu
from jax.experimental.pallas import tpu_sc as plsc

_N = 10000
_E = 320000
_H = 128
_G = 64
_C = 8

_NC = 2
_NS = 16
_NW = _NC * _NS
_WIN = 125
_WPW = _E // (_NW * _WIN)
_HPW = _WPW // 2
_CH = 1000
_NCH = _N // _CH
_TCH = (_NCH + _NS - 1) // _NS
_DW = _H


def _zero_acc(zero_hbm, acc, sid, sem):
    @pl.loop(0, _TCH)
    def _zero(t):
        j = sid + t * _NS

        @pl.when(j < _NCH)
        def _():
            r0 = pl.multiple_of(j * _CH, 8)
            pltpu.async_copy(zero_hbm, acc.at[pl.ds(r0, _CH)], sem)

    @pl.loop(0, _TCH)
    def _zwait(t):
        j = sid + t * _NS

        @pl.when(j < _NCH)
        def _():
            r0 = pl.multiple_of(j * _CH, 8)
            pltpu.make_async_copy(zero_hbm, acc.at[pl.ds(r0, _CH)], sem).wait()


def _write_partial(acc, out_hbm, cid, sid, sem):
    @pl.loop(0, _TCH)
    def _out(t):
        j = sid + t * _NS

        @pl.when(j < _NCH)
        def _():
            r0 = pl.multiple_of(j * _CH, 8)
            pltpu.async_copy(acc.at[pl.ds(r0, _CH)],
                             out_hbm.at[pl.ds(cid * _N + r0, _CH)], sem)

    @pl.loop(0, _TCH)
    def _owait(t):
        j = sid + t * _NS

        @pl.when(j < _NCH)
        def _():
            r0 = pl.multiple_of(j * _CH, 8)
            pltpu.make_async_copy(acc.at[pl.ds(r0, _CH)],
                                  out_hbm.at[pl.ds(cid * _N + r0, _CH)],
                                  sem).wait()


def _sc_scatter_body(z_hbm, src_hbm, dst_hbm, zero_hbm, out_hbm,
                     sidx, didx, rows0, rows1, acc,
                     gsem0, gsem1, zsem):
    cid = lax.axis_index("c")
    sid = lax.axis_index("s")
    wid = sid * _NC + cid
    base = pl.multiple_of(wid * _WPW, 8)

    pltpu.async_copy(src_hbm.at[pl.ds(base, _HPW)], sidx, gsem0)
    pltpu.async_copy(dst_hbm.at[pl.ds(base, _HPW)], didx, gsem1)

    _zero_acc(zero_hbm, acc, sid, zsem)
    pltpu.make_async_copy(src_hbm.at[pl.ds(base, _HPW)], sidx, gsem0).wait()
    pltpu.make_async_copy(dst_hbm.at[pl.ds(base, _HPW)], didx, gsem1).wait()
    plsc.subcore_barrier()

    pltpu.async_copy(z_hbm.at[sidx.at[0]], rows0, gsem0)
    pltpu.async_copy(z_hbm.at[sidx.at[1]], rows1, gsem1)

    @pl.loop(0, _WPW // 2)
    def _edges(t):
        j = 2 * t

        @pl.when(t == _HPW // 2)
        def _():
            pltpu.sync_copy(src_hbm.at[pl.ds(base + _HPW, _HPW)], sidx)
            pltpu.sync_copy(dst_hbm.at[pl.ds(base + _HPW, _HPW)], didx)
            pltpu.async_copy(z_hbm.at[sidx.at[0]], rows0, gsem0)
            pltpu.async_copy(z_hbm.at[sidx.at[1]], rows1, gsem1)

        jl = lax.rem(j, _HPW)
        not_last = jnp.logical_and(t != _HPW // 2 - 1, t != _WPW // 2 - 1)

        pltpu.make_async_copy(z_hbm.at[sidx.at[jl]], rows0, gsem0).wait()
        pltpu.sync_copy(rows0, acc.at[didx.at[jl]], add=True)

        @pl.when(not_last)
        def _():
            pltpu.async_copy(z_hbm.at[sidx.at[jl + 2]], rows0, gsem0)

        pltpu.make_async_copy(z_hbm.at[sidx.at[jl + 1]], rows1, gsem1).wait()
        pltpu.sync_copy(rows1, acc.at[didx.at[jl + 1]], add=True)

        @pl.when(not_last)
        def _():
            pltpu.async_copy(z_hbm.at[sidx.at[jl + 3]], rows1, gsem1)

    plsc.subcore_barrier()
    _write_partial(acc, out_hbm, cid, sid, gsem0)


def _sc_scatter(z, src2, dst2, zero_chunk):
    f = pl.kernel(
        _sc_scatter_body,
        out_type=jax.ShapeDtypeStruct((2 * _N, _H), jnp.float32),
        mesh=plsc.VectorSubcoreMesh(core_axis_name="c", subcore_axis_name="s"),
        scratch_types=[
            pltpu.VMEM((_HPW, _WIN), jnp.int32),
            pltpu.VMEM((_HPW, _WIN), jnp.int32),
            pltpu.VMEM((_WIN, _H), jnp.float32),
            pltpu.VMEM((_WIN, _H), jnp.float32),
            pltpu.VMEM_SHARED((_N, _H), jnp.float32),
            pltpu.SemaphoreType.DMA,
            pltpu.SemaphoreType.DMA,
            pltpu.SemaphoreType.DMA,
        ],
    )
    return f(z, src2, dst2, zero_chunk)


def _sc_degree_body(dst_hbm, ones_hbm, zero_hbm, out_hbm,
                    didx, ones_v, acc, sem0, zsem):
    cid = lax.axis_index("c")
    sid = lax.axis_index("s")
    wid = sid * _NC + cid
    base = pl.multiple_of(wid * _WPW, 8)

    pltpu.async_copy(dst_hbm.at[pl.ds(base, _WPW)], didx, sem0)
    pltpu.sync_copy(ones_hbm, ones_v)

    _zero_acc(zero_hbm, acc, sid, zsem)
    pltpu.make_async_copy(dst_hbm.at[pl.ds(base, _WPW)], didx, sem0).wait()
    plsc.subcore_barrier()

    @pl.loop(0, _WPW // 16)
    def _edges(b):
        hs = [pltpu.async_copy(ones_v, acc.at[didx.at[b * 16 + k]], zsem,
                               add=True)
              for k in range(16)]
        for h in hs:
            h.wait()

    plsc.subcore_barrier()
    _write_partial(acc, out_hbm, cid, sid, sem0)


def _sc_degree(dst2, ones_win, zero_chunk):
    f = pl.kernel(
        _sc_degree_body,
        out_type=jax.ShapeDtypeStruct((2 * _N, _DW), jnp.float32),
        mesh=plsc.VectorSubcoreMesh(core_axis_name="c", subcore_axis_name="s"),
        scratch_types=[
            pltpu.VMEM((_WPW, _WIN), jnp.int32),
            pltpu.VMEM((_WIN, _DW), jnp.float32),
            pltpu.VMEM_SHARED((_N, _DW), jnp.float32),
            pltpu.SemaphoreType.DMA,
            pltpu.SemaphoreType.DMA,
        ],
    )
    return f(dst2, ones_win, zero_chunk)


_R = 2000


def _tmm_body(x_ref, w_ref, y_ref):
    y_ref[...] = jnp.dot(x_ref[...], w_ref[...],
                         preferred_element_type=jnp.float32)


def _tc_matmul(x, w0):
    return pl.pallas_call(
        _tmm_body,
        grid=(_N // _R,),
        in_specs=[
            pl.BlockSpec((_R, _H), lambda i: (i, 0)),
            pl.BlockSpec((_H, _H), lambda i: (0, 0)),
        ],
        out_specs=pl.BlockSpec((_R, _H), lambda i: (i, 0)),
        out_shape=jax.ShapeDtypeStruct((_N, _H), jnp.float32),
    )(x, w0)


_NB = _N // _R


def _t0_body(y_ref, d0_ref, d1_ref, z_ref, dinv_ref):
    d = d0_ref[...][:, :1] + d1_ref[...][:, :1]
    dinv = jnp.broadcast_to(lax.rsqrt(1.0 + d), (_R, _H))
    dinv_ref[...] = dinv
    z_ref[...] = y_ref[...] * dinv


def _tc_first(y, degp):
    return pl.pallas_call(
        _t0_body,
        grid=(_NB,),
        in_specs=[
            pl.BlockSpec((_R, _H), lambda i: (i, 0)),
            pl.BlockSpec((_R, _DW), lambda i: (i, 0)),
            pl.BlockSpec((_R, _DW), lambda i: (i + _NB, 0)),
        ],
        out_specs=[
            pl.BlockSpec((_R, _H), lambda i: (i, 0)),
            pl.BlockSpec((_R, _H), lambda i: (i, 0)),
        ],
        out_shape=[
            jax.ShapeDtypeStruct((_N, _H), jnp.float32),
            jax.ShapeDtypeStruct((_N, _H), jnp.float32),
        ],
    )(y, degp, degp)


def _tmid_body(p0_ref, p1_ref, z_ref, dinv_ref, b_ref, w_ref, zo_ref):
    dinv = dinv_ref[...]
    h = jnp.maximum(
        dinv * (p0_ref[...] + p1_ref[...] + z_ref[...]) + b_ref[...], 0.0)
    zo_ref[...] = jnp.dot(h, w_ref[...],
                          preferred_element_type=jnp.float32) * dinv


def _tc_mid(p, z, dinv, b, w):
    return pl.pallas_call(
        _tmid_body,
        grid=(_NB,),
        in_specs=[
            pl.BlockSpec((_R, _H), lambda i: (i, 0)),
            pl.BlockSpec((_R, _H), lambda i: (i + _NB, 0)),
            pl.BlockSpec((_R, _H), lambda i: (i, 0)),
            pl.BlockSpec((_R, _H), lambda i: (i, 0)),
            pl.BlockSpec((1, _H), lambda i: (0, 0)),
            pl.BlockSpec((_H, _H), lambda i: (0, 0)),
        ],
        out_specs=pl.BlockSpec((_R, _H), lambda i: (i, 0)),
        out_shape=jax.ShapeDtypeStruct((_N, _H), jnp.float32),
    )(p, p, z, dinv, b, w)


def _t4_body(p0_ref, p1_ref, z_ref, dinv_ref, b_ref, batch_ref, wc_ref,
             bc_ref, sums_ref, cnt_ref, out_ref):
    i = pl.program_id(0)
    h = dinv_ref[...] * (p0_ref[...] + p1_ref[...] + z_ref[...]) + b_ref[...]
    gids = lax.broadcasted_iota(jnp.int32, (_R, _G), 1)
    m = (batch_ref[...] == gids).astype(jnp.float32)
    s = lax.dot_general(m, h, (((0,), (0,)), ((), ())),
                        preferred_element_type=jnp.float32)
    c = jnp.broadcast_to(jnp.sum(m, axis=0)[:, None], (_G, _H))

    @pl.when(i == 0)
    def _():
        sums_ref[...] = s
        cnt_ref[...] = c

    @pl.when(i > 0)
    def _():
        sums_ref[...] += s
        cnt_ref[...] += c

    @pl.when(i == _NB - 1)
    def _():
        mean = sums_ref[...] / jnp.maximum(cnt_ref[...], 1.0)
        out_ref[...] = jnp.dot(mean, wc_ref[...],
                               preferred_element_type=jnp.float32) + bc_ref[...]


def _tc_pool(p, z, dinv, b, batch2d, wc_pad, bc_pad):
    outs = pl.pallas_call(
        _t4_body,
        grid=(_NB,),
        in_specs=[
            pl.BlockSpec((_R, _H), lambda i: (i, 0)),
            pl.BlockSpec((_R, _H), lambda i: (i + _NB, 0)),
            pl.BlockSpec((_R, _H), lambda i: (i, 0)),
            pl.BlockSpec((_R, _H), lambda i: (i, 0)),
            pl.BlockSpec((1, _H), lambda i: (0, 0)),
            pl.BlockSpec((_R, 1), lambda i: (i, 0)),
            pl.BlockSpec((_H, _H), lambda i: (0, 0)),
            pl.BlockSpec((1, _H), lambda i: (0, 0)),
        ],
        out_specs=[
            pl.BlockSpec((_G, _H), lambda i: (0, 0)),
            pl.BlockSpec((_G, _H), lambda i: (0, 0)),
            pl.BlockSpec((_G, _H), lambda i: (0, 0)),
        ],
        out_shape=[
            jax.ShapeDtypeStruct((_G, _H), jnp.float32),
            jax.ShapeDtypeStruct((_G, _H), jnp.float32),
            jax.ShapeDtypeStruct((_G, _H), jnp.float32),
        ],
    )(p, p, z, dinv, b, batch2d, wc_pad, bc_pad)
    return outs[2]


def kernel(x, edge_index, batch, W0, b0, W1, b1, W2, b2, W3, b3, Wc, bc):
    src2 = edge_index[0].reshape(_E // _WIN, _WIN)
    dst2 = edge_index[1].reshape(_E // _WIN, _WIN)
    zero_chunk = jnp.zeros((_CH, _H), jnp.float32)
    ones_win = jnp.ones((_WIN, _DW), jnp.float32)

    degp = _sc_degree(dst2, ones_win, jnp.zeros((_CH, _DW), jnp.float32))
    y0 = _tc_matmul(x, W0)

    z, dinv = _tc_first(y0, degp)

    for b, w in ((b0, W1), (b1, W2), (b2, W3)):
        p = _sc_scatter(z, src2, dst2, zero_chunk)
        z = _tc_mid(p, z, dinv, b.reshape(1, _H), w)

    p = _sc_scatter(z, src2, dst2, zero_chunk)

    wc_pad = jnp.zeros((_H, _H), jnp.float32).at[:, :_C].set(Wc)
    bc_pad = jnp.zeros((1, _H), jnp.float32).at[0, :_C].set(bc)
    out = _tc_pool(p, z, dinv, b3.reshape(1, _H),
                   batch.reshape(_N, 1), wc_pad, bc_pad)
    return out[:, :_C]
